# baseline (speedup 1.0000x reference)
import functools

import jax
import jax.numpy as jnp
from jax import lax
from jax.experimental import pallas as pl
from jax.experimental.pallas import tpu as pltpu
from jax.experimental.pallas import tpu_sc as plsc

NC = 2
NS = 16
NW = NC * NS
LANES = 16
NPAD = 10240
RPT = NPAD // NS
K = 128

_MESH = plsc.VectorSubcoreMesh(core_axis_name="c", subcore_axis_name="s")
_PARAMS = pltpu.CompilerParams(use_tc_tiling_on_sc=False)
_PARAMS_NL = pltpu.CompilerParams(use_tc_tiling_on_sc=False,
                                  needs_layout_passes=False)


def _fill1(ref, val, n):
    def body(i, carry):
        ref[pl.ds(i * LANES, LANES)] = jnp.full((LANES,), val, ref.dtype)
        return carry
    lax.fori_loop(0, n // LANES, body, 0)


def _rsqrt_sc(d):
    i = lax.bitcast_convert_type(d, jnp.int32)
    y = lax.bitcast_convert_type(jnp.int32(0x5F3759DF) - (i >> 1), jnp.float32)
    for _ in range(3):
        y = y * (1.5 - 0.5 * d * y * y)
    return y


def _make_conv1_kernel(nch, h):
    @functools.partial(
        pl.kernel,
        out_type=(jax.ShapeDtypeStruct((NC, NPAD, h), jnp.float32),
                  jax.ShapeDtypeStruct((NPAD,), jnp.float32)),
        mesh=_MESH,
        scratch_types=[
            pltpu.VMEM((nch, K), jnp.int32),
            pltpu.VMEM((nch, K), jnp.int32),
            pltpu.VMEM((2, K, h), jnp.float32),
            pltpu.VMEM((K,), jnp.float32),
            pltpu.VMEM((RPT,), jnp.float32),
            pltpu.VMEM((NPAD,), jnp.float32),
            pltpu.VMEM((RPT, h), jnp.float32),
            pltpu.VMEM_SHARED((NPAD,), jnp.float32),
            pltpu.VMEM_SHARED((NPAD,), jnp.float32),
            pltpu.VMEM_SHARED((NPAD, h), jnp.float32),
            pltpu.SemaphoreType.DMA,
            pltpu.SemaphoreType.DMA,
        ],
        compiler_params=_PARAMS_NL,
    )
    def conv1_kernel(tab_hbm, src_hbm, dst_hbm, zero_hbm, out_hbm, dinv_hbm,
                     idx_s, idx_d, rows, ones_v, dslice, dinv_v, zv,
                     dacc, dshare, acc, gsem, ssem):
        cid = lax.axis_index("c")
        sid = lax.axis_index("s")
        wid = cid * NS + sid

        _fill1(ones_v, 1.0, K)
        _fill1(dslice, 0.0, RPT)
        pltpu.sync_copy(dslice, dacc.at[pl.ds(sid * RPT, RPT)])
        pltpu.sync_copy(zero_hbm, zv)
        pltpu.sync_copy(zv, acc.at[pl.ds(sid * RPT, RPT)])
        plsc.subcore_barrier()

        def deg_worker(w):
            pltpu.sync_copy(dst_hbm.at[w], idx_d)

            def chunk(i, carry):
                pltpu.sync_copy(ones_v, dacc.at[idx_d.at[i]], add=True)
                return carry

            lax.fori_loop(0, nch, chunk, 0)

        deg_worker(2 * sid)
        deg_worker(2 * sid + 1)
        plsc.subcore_barrier()

        pltpu.sync_copy(dacc.at[pl.ds(sid * RPT, RPT)], dslice)

        def dinv_blk(k, carry):
            d = dslice[pl.ds(k * LANES, LANES)] + 1.0
            dslice[pl.ds(k * LANES, LANES)] = _rsqrt_sc(d)
            return carry

        lax.fori_loop(0, RPT // LANES, dinv_blk, 0)
        pltpu.sync_copy(dslice, dshare.at[pl.ds(sid * RPT, RPT)])

        @pl.when(cid == 0)
        def _():
            pltpu.sync_copy(dslice, dinv_hbm.at[pl.ds(sid * RPT, RPT)])

        plsc.subcore_barrier()
        pltpu.sync_copy(dshare, dinv_v)

        pltpu.sync_copy(src_hbm.at[wid], idx_s)
        pltpu.sync_copy(dst_hbm.at[wid], idx_d)
        pltpu.async_copy(tab_hbm.at[idx_s.at[0]], rows.at[0], gsem)

        def substep(i, b):
            pltpu.make_async_copy(tab_hbm.at[idx_s.at[i]], rows.at[b],
                                  gsem).wait()

            @pl.when(i + 1 < nch)
            def _():
                pltpu.async_copy(tab_hbm.at[idx_s.at[i + 1]],
                                 rows.at[1 - b], gsem)

            for g in range(K // LANES):
                sv = idx_s[i, pl.ds(g * LANES, LANES)]
                dv = plsc.load_gather(dinv_v, [sv])
                for r in range(LANES):
                    rr = g * LANES + r
                    rows[b, rr, :] = rows[b, rr, :] * dv[r]

            @pl.when(i >= 1)
            def _():
                pltpu.make_async_copy(rows.at[1 - b],
                                      acc.at[idx_d.at[i - 1]], ssem).wait()

            pltpu.async_copy(rows.at[b], acc.at[idx_d.at[i]], ssem, add=True)

        def pair(j, carry):
            substep(2 * j, 0)
            substep(2 * j + 1, 1)
            return carry

        lax.fori_loop(0, nch // 2, pair, 0)
        pltpu.make_async_copy(rows.at[1], acc.at[idx_d.at[nch - 1]],
                              ssem).wait()
        plsc.subcore_barrier()
        pltpu.sync_copy(acc.at[pl.ds(sid * RPT, RPT)],
                        out_hbm.at[cid, pl.ds(sid * RPT, RPT)])

    return conv1_kernel


def _make_scatter_kernel(nch, width):
    @functools.partial(
        pl.kernel,
        out_type=jax.ShapeDtypeStruct((NC, NPAD, width), jnp.float32),
        mesh=_MESH,
        scratch_types=[
            pltpu.VMEM((nch, K), jnp.int32),
            pltpu.VMEM((nch, K), jnp.int32),
            pltpu.VMEM((2, K, width), jnp.float32),
            pltpu.VMEM((RPT, width), jnp.float32),
            pltpu.VMEM_SHARED((NPAD, width), jnp.float32),
            pltpu.SemaphoreType.DMA,
        ],
        compiler_params=_PARAMS,
    )
    def scatter_kernel(tab_hbm, src_hbm, dst_hbm, zero_hbm, out_hbm,
                       idx_s, idx_d, rows, zv, acc, sem):
        cid = lax.axis_index("c")
        sid = lax.axis_index("s")
        wid = cid * NS + sid
        pltpu.sync_copy(src_hbm.at[wid], idx_s)
        pltpu.sync_copy(dst_hbm.at[wid], idx_d)
        pltpu.sync_copy(zero_hbm, zv)
        pltpu.sync_copy(zv, acc.at[pl.ds(sid * RPT, RPT)])
        plsc.subcore_barrier()

        pltpu.async_copy(tab_hbm.at[idx_s.at[0]], rows.at[0], sem)

        def chunk(i, carry):
            cur = lax.rem(i, 2)
            pltpu.make_async_copy(tab_hbm.at[idx_s.at[i]], rows.at[cur],
                                  sem).wait()

            @pl.when(i + 1 < nch)
            def _():
                pltpu.async_copy(tab_hbm.at[idx_s.at[i + 1]],
                                 rows.at[1 - cur], sem)

            pltpu.sync_copy(rows.at[cur], acc.at[idx_d.at[i]], add=True)
            return carry

        lax.fori_loop(0, nch, chunk, 0)
        plsc.subcore_barrier()
        pltpu.sync_copy(acc.at[pl.ds(sid * RPT, RPT)],
                        out_hbm.at[cid, pl.ds(sid * RPT, RPT)])

    return scatter_kernel


def _tc_prep_body(x_ref, v_ref, gv_ref, bv_ref, bfe_ref, W1fe_ref, W1v_ref,
                  hw1_ref):
    xx = x_ref[...]
    n = xx.shape[0]
    h = hw1_ref.shape[1]
    mx = jnp.mean(xx, axis=0, keepdims=True)
    vx = jnp.mean((xx - mx) ** 2, axis=0, keepdims=True)
    v = v_ref[...]
    s = gv_ref[...] * v * lax.rsqrt(v * v * vx + 1e-5)
    o = bv_ref[...] - s * mx
    Kc = (jnp.dot(bfe_ref[...], W1fe_ref[...], preferred_element_type=jnp.float32)
          + jnp.dot(o, W1v_ref[...], preferred_element_type=jnp.float32))
    hw1_ref[:n, :] = (jnp.dot(xx * s, W1v_ref[...],
                              preferred_element_type=jnp.float32) + Kc)
    hw1_ref[n:, :] = jnp.zeros((hw1_ref.shape[0] - n, h), jnp.float32)


def _tc_mid_body(a10_ref, a11_ref, hw1_ref, dinv_ref, b1_ref, g2_ref):
    n = dinv_ref.shape[0]
    h = g2_ref.shape[1]
    dinv = dinv_ref[...]
    pre = ((a10_ref[:n] + a11_ref[:n]) * dinv
           + hw1_ref[:n] * (dinv * dinv) + b1_ref[...])
    out1 = jnp.maximum(pre, 0.0)
    g2_ref[:n, :] = out1 * dinv
    g2_ref[n:, :] = jnp.zeros((g2_ref.shape[0] - n, h), jnp.float32)


def _tc_final_body(a20_ref, a21_ref, g2_ref, dinv_ref, W2_ref, b2_ref, out_ref):
    n, c = out_ref.shape
    hh = g2_ref.shape[1]
    pre = (a20_ref[:n, :hh] + a21_ref[:n, :hh] + g2_ref[:n, :hh]) * dinv_ref[...]
    h = jnp.dot(pre, W2_ref[...], preferred_element_type=jnp.float32) + b2_ref[...]
    m = jnp.max(h, axis=1, keepdims=True)
    lse = jnp.log(jnp.sum(jnp.exp(h - m), axis=1, keepdims=True))
    out_ref[...] = h - m - lse


def kernel(x, edge_index, feat_emb, val_emb, gamma, beta, W1, b1, W2, b2):
    N, D = x.shape
    E = edge_index.shape[1]
    FE = feat_emb.shape[1]
    CH = FE + val_emb.shape[1]
    H = W1.shape[1]
    C = W2.shape[1]

    nch = -(-E // (NW * K))
    nch += nch % 2
    EP = NW * K * nch
    pad = N + (jnp.arange(EP - E, dtype=jnp.int32) % (NPAD - N))
    src = jnp.concatenate([edge_index[0], pad]).reshape(NW, nch, K)
    dst = jnp.concatenate([edge_index[1], pad]).reshape(NW, nch, K)

    g5 = gamma.reshape(D, CH)
    b5 = beta.reshape(D, CH)
    W1r = W1.reshape(D, CH, H)
    gv = g5[:, FE].reshape(1, D)
    bv = b5[:, FE].reshape(1, D)
    v = val_emb[:, 0].reshape(1, D)
    W1v = W1r[:, FE, :]
    W1fe = W1r[:, :FE, :].reshape(FE * D, H)
    bfe = b5[:, :FE].reshape(1, FE * D)
    zero_h = jnp.zeros((RPT, H), jnp.float32)

    hw1 = pl.pallas_call(
        _tc_prep_body,
        out_shape=jax.ShapeDtypeStruct((NPAD, H), jnp.float32),
    )(x, v, gv, bv, bfe, W1fe, W1v)

    acc1, dinv_all = _make_conv1_kernel(nch, H)(hw1, src, dst, zero_h)
    dinv = dinv_all[:N].reshape(N, 1)

    g2 = pl.pallas_call(
        _tc_mid_body,
        out_shape=jax.ShapeDtypeStruct((NPAD, H), jnp.float32),
    )(acc1[0], acc1[1], hw1, dinv, b1.reshape(1, H))

    acc2 = _make_scatter_kernel(nch, H)(g2, src, dst, zero_h)

    out = pl.pallas_call(
        _tc_final_body,
        out_shape=jax.ShapeDtypeStruct((N, C), jnp.float32),
    )(acc2[0], acc2[1], g2, dinv, W2, b2.reshape(1, C))
    return out


# speedup vs baseline: 1.0193x; 1.0193x over previous
import functools

import jax
import jax.numpy as jnp
from jax import lax
from jax.experimental import pallas as pl
from jax.experimental.pallas import tpu as pltpu
from jax.experimental.pallas import tpu_sc as plsc

NC = 2
NS = 16
NW = NC * NS
LANES = 16
NPAD = 10240
RPT = NPAD // NS
K = 128

_MESH = plsc.VectorSubcoreMesh(core_axis_name="c", subcore_axis_name="s")
_PARAMS = pltpu.CompilerParams(use_tc_tiling_on_sc=False)


def _fill1(ref, val, n):
    def body(i, carry):
        ref[pl.ds(i * LANES, LANES)] = jnp.full((LANES,), val, ref.dtype)
        return carry
    lax.fori_loop(0, n // LANES, body, 0)


def _make_deg_kernel(nch):
    @functools.partial(
        pl.kernel,
        out_type=jax.ShapeDtypeStruct((NC, NPAD), jnp.float32),
        mesh=_MESH,
        scratch_types=[
            pltpu.VMEM((nch, K), jnp.int32),
            pltpu.VMEM((K,), jnp.float32),
            pltpu.VMEM((RPT,), jnp.float32),
            pltpu.VMEM_SHARED((NPAD,), jnp.float32),
        ],
        compiler_params=_PARAMS,
    )
    def deg_kernel(dst_hbm, out_hbm, idx_v, ones_v, zero_v, acc):
        cid = lax.axis_index("c")
        sid = lax.axis_index("s")
        wid = cid * NS + sid
        pltpu.sync_copy(dst_hbm.at[wid], idx_v)
        _fill1(ones_v, 1.0, K)
        _fill1(zero_v, 0.0, RPT)
        pltpu.sync_copy(zero_v, acc.at[pl.ds(sid * RPT, RPT)])
        plsc.subcore_barrier()

        def chunk(i, carry):
            pltpu.sync_copy(ones_v, acc.at[idx_v.at[i]], add=True)
            return carry

        lax.fori_loop(0, nch, chunk, 0)
        plsc.subcore_barrier()
        pltpu.sync_copy(acc.at[pl.ds(sid * RPT, RPT)],
                        out_hbm.at[cid, pl.ds(sid * RPT, RPT)])

    return deg_kernel


def _make_scatter_kernel(nch, width):
    @functools.partial(
        pl.kernel,
        out_type=jax.ShapeDtypeStruct((NC, NPAD, width), jnp.float32),
        mesh=_MESH,
        scratch_types=[
            pltpu.VMEM((nch, K), jnp.int32),
            pltpu.VMEM((nch, K), jnp.int32),
            pltpu.VMEM((2, K, width), jnp.float32),
            pltpu.VMEM((RPT, width), jnp.float32),
            pltpu.VMEM_SHARED((NPAD, width), jnp.float32),
            pltpu.SemaphoreType.DMA,
        ],
        compiler_params=_PARAMS,
    )
    def scatter_kernel(tab_hbm, src_hbm, dst_hbm, zero_hbm, out_hbm,
                       idx_s, idx_d, rows, zv, acc, sem):
        cid = lax.axis_index("c")
        sid = lax.axis_index("s")
        wid = cid * NS + sid
        pltpu.sync_copy(src_hbm.at[wid], idx_s)
        pltpu.sync_copy(dst_hbm.at[wid], idx_d)
        pltpu.sync_copy(zero_hbm, zv)
        pltpu.sync_copy(zv, acc.at[pl.ds(sid * RPT, RPT)])
        plsc.subcore_barrier()

        pltpu.async_copy(tab_hbm.at[idx_s.at[0]], rows.at[0], sem)

        def chunk(i, carry):
            cur = lax.rem(i, 2)
            pltpu.make_async_copy(tab_hbm.at[idx_s.at[i]], rows.at[cur],
                                  sem).wait()

            @pl.when(i + 1 < nch)
            def _():
                pltpu.async_copy(tab_hbm.at[idx_s.at[i + 1]],
                                 rows.at[1 - cur], sem)

            pltpu.sync_copy(rows.at[cur], acc.at[idx_d.at[i]], add=True)
            return carry

        lax.fori_loop(0, nch, chunk, 0)
        plsc.subcore_barrier()
        pltpu.sync_copy(acc.at[pl.ds(sid * RPT, RPT)],
                        out_hbm.at[cid, pl.ds(sid * RPT, RPT)])

    return scatter_kernel


def _rsqrt_sc(d):
    i = lax.bitcast_convert_type(d, jnp.int32)
    y = lax.bitcast_convert_type(jnp.int32(0x5F3759DF) - (i >> 1), jnp.float32)
    for _ in range(3):
        y = y * (1.5 - 0.5 * d * y * y)
    return y


def _make_conv2_kernel(nch, h):
    @functools.partial(
        pl.kernel,
        out_type=(jax.ShapeDtypeStruct((NC, NPAD, h), jnp.float32),
                  jax.ShapeDtypeStruct((NC, NPAD, h), jnp.float32)),
        mesh=_MESH,
        scratch_types=[
            pltpu.VMEM((nch, K), jnp.int32),
            pltpu.VMEM((nch, K), jnp.int32),
            pltpu.VMEM((2, K, h), jnp.float32),
            pltpu.VMEM((RPT, h), jnp.float32),
            pltpu.VMEM((RPT, h), jnp.float32),
            pltpu.VMEM((RPT, h), jnp.float32),
            pltpu.VMEM((RPT,), jnp.float32),
            pltpu.VMEM((RPT,), jnp.float32),
            pltpu.VMEM((1, h), jnp.float32),
            pltpu.VMEM((RPT, h), jnp.float32),
            pltpu.VMEM_SHARED((NPAD, h), jnp.float32),
            pltpu.SemaphoreType.DMA,
        ],
        compiler_params=_PARAMS,
    )
    def conv2_kernel(a10_hbm, a11_hbm, g1_hbm, d0_hbm, d1_hbm, b1_hbm,
                     src_hbm, dst_hbm, zero_hbm, g2_hbm, out_hbm,
                     idx_s, idx_d, rows, a0v, a1v, g1v, dv, d1v, b1v, zv,
                     acc, sem):
        cid = lax.axis_index("c")
        sid = lax.axis_index("s")
        wid = cid * NS + sid
        sl = pl.ds(sid * RPT, RPT)
        pltpu.sync_copy(src_hbm.at[wid], idx_s)
        pltpu.sync_copy(dst_hbm.at[wid], idx_d)
        pltpu.sync_copy(a10_hbm.at[sl], a0v)
        pltpu.sync_copy(a11_hbm.at[sl], a1v)
        pltpu.sync_copy(g1_hbm.at[sl], g1v)
        pltpu.sync_copy(d0_hbm.at[sl], dv)
        pltpu.sync_copy(d1_hbm.at[sl], d1v)
        pltpu.sync_copy(b1_hbm, b1v)
        pltpu.sync_copy(zero_hbm, zv)
        pltpu.sync_copy(zv, acc.at[sl])

        def dinv_blk(k, carry):
            d = dv[pl.ds(k * LANES, LANES)] + d1v[pl.ds(k * LANES, LANES)] + 1.0
            dv[pl.ds(k * LANES, LANES)] = _rsqrt_sc(d)
            return carry

        lax.fori_loop(0, RPT // LANES, dinv_blk, 0)

        b1row = b1v[0, :]

        def g2_blk(k, carry):
            dv16 = dv[pl.ds(k * LANES, LANES)]
            for r in range(LANES):
                rr = k * LANES + r
                row = (a0v[rr, :] + a1v[rr, :] + g1v[rr, :]) * dv16[r] + b1row
                a0v[rr, :] = jnp.maximum(row, 0.0) * dv16[r]
            return carry

        lax.fori_loop(0, RPT // LANES, g2_blk, 0)
        pltpu.sync_copy(a0v, g2_hbm.at[cid, sl])
        plsc.subcore_barrier()

        tab = g2_hbm.at[cid]
        pltpu.async_copy(tab.at[idx_s.at[0]], rows.at[0], sem)

        def chunk(i, carry):
            cur = lax.rem(i, 2)
            pltpu.make_async_copy(tab.at[idx_s.at[i]], rows.at[cur],
                                  sem).wait()

            @pl.when(i + 1 < nch)
            def _():
                pltpu.async_copy(tab.at[idx_s.at[i + 1]],
                                 rows.at[1 - cur], sem)

            pltpu.sync_copy(rows.at[cur], acc.at[idx_d.at[i]], add=True)
            return carry

        lax.fori_loop(0, nch, chunk, 0)
        plsc.subcore_barrier()
        pltpu.sync_copy(acc.at[sl], out_hbm.at[cid, sl])

    return conv2_kernel


def _tc_prep_body(x_ref, v_ref, gv_ref, bv_ref, bfe_ref, W1fe_ref, W1v_ref,
                  d0_ref, d1_ref, g1_ref, dinv_ref):
    xx = x_ref[...]
    n = xx.shape[0]
    h = g1_ref.shape[1]
    mx = jnp.mean(xx, axis=0, keepdims=True)
    vx = jnp.mean((xx - mx) ** 2, axis=0, keepdims=True)
    v = v_ref[...]
    s = gv_ref[...] * v * lax.rsqrt(v * v * vx + 1e-5)
    o = bv_ref[...] - s * mx
    Kc = (jnp.dot(bfe_ref[...], W1fe_ref[...], preferred_element_type=jnp.float32)
          + jnp.dot(o, W1v_ref[...], preferred_element_type=jnp.float32))
    hw1 = jnp.dot(xx * s, W1v_ref[...], preferred_element_type=jnp.float32) + Kc
    deg = d0_ref[:n] + d1_ref[:n] + 1.0
    dinv = lax.rsqrt(deg)
    dinv_ref[...] = dinv
    g1_ref[:n, :] = hw1 * dinv
    g1_ref[n:, :] = jnp.zeros((g1_ref.shape[0] - n, h), jnp.float32)


def _tc_final_body(a20_ref, a21_ref, g2_ref, dinv_ref, W2_ref, b2_ref, out_ref):
    n, c = out_ref.shape
    hh = g2_ref.shape[1]
    pre = (a20_ref[:n, :hh] + a21_ref[:n, :hh] + g2_ref[:n, :hh]) * dinv_ref[...]
    h = jnp.dot(pre, W2_ref[...], preferred_element_type=jnp.float32) + b2_ref[...]
    m = jnp.max(h, axis=1, keepdims=True)
    lse = jnp.log(jnp.sum(jnp.exp(h - m), axis=1, keepdims=True))
    out_ref[...] = h - m - lse


def kernel(x, edge_index, feat_emb, val_emb, gamma, beta, W1, b1, W2, b2):
    N, D = x.shape
    E = edge_index.shape[1]
    FE = feat_emb.shape[1]
    CH = FE + val_emb.shape[1]
    H = W1.shape[1]
    C = W2.shape[1]

    nch = -(-E // (NW * K))
    EP = NW * K * nch
    pad = N + (jnp.arange(EP - E, dtype=jnp.int32) % (NPAD - N))
    src = jnp.concatenate([edge_index[0], pad]).reshape(NW, nch, K)
    dst = jnp.concatenate([edge_index[1], pad]).reshape(NW, nch, K)

    g5 = gamma.reshape(D, CH)
    b5 = beta.reshape(D, CH)
    W1r = W1.reshape(D, CH, H)
    gv = g5[:, FE].reshape(1, D)
    bv = b5[:, FE].reshape(1, D)
    v = val_emb[:, 0].reshape(1, D)
    W1v = W1r[:, FE, :]
    W1fe = W1r[:, :FE, :].reshape(FE * D, H)
    bfe = b5[:, :FE].reshape(1, FE * D)
    zero_h = jnp.zeros((RPT, H), jnp.float32)

    degp = _make_deg_kernel(nch)(dst)
    d0 = degp[0].reshape(NPAD, 1)
    d1 = degp[1].reshape(NPAD, 1)

    g1, dinv = pl.pallas_call(
        _tc_prep_body,
        out_shape=(jax.ShapeDtypeStruct((NPAD, H), jnp.float32),
                   jax.ShapeDtypeStruct((N, 1), jnp.float32)),
    )(x, v, gv, bv, bfe, W1fe, W1v, d0, d1)

    acc1 = _make_scatter_kernel(nch, H)(g1, src, dst, zero_h)

    g2_all, acc2 = _make_conv2_kernel(nch, H)(
        acc1[0], acc1[1], g1, degp[0], degp[1], b1.reshape(1, H),
        src, dst, zero_h)
    g2 = g2_all[0]

    out = pl.pallas_call(
        _tc_final_body,
        out_shape=jax.ShapeDtypeStruct((N, C), jnp.float32),
    )(acc2[0], acc2[1], g2, dinv, W2, b2.reshape(1, C))
    return out


# speedup vs baseline: 1.0397x; 1.0200x over previous
import functools

import jax
import jax.numpy as jnp
from jax import lax
from jax.experimental import pallas as pl
from jax.experimental.pallas import tpu as pltpu
from jax.experimental.pallas import tpu_sc as plsc

NC = 2
NS = 16
NW = NC * NS
LANES = 16
NPAD = 10240
RPT = NPAD // NS
K = 128

_MESH = plsc.VectorSubcoreMesh(core_axis_name="c", subcore_axis_name="s")
_PARAMS = pltpu.CompilerParams(use_tc_tiling_on_sc=False)


def _fill1(ref, val, n):
    def body(i, carry):
        ref[pl.ds(i * LANES, LANES)] = jnp.full((LANES,), val, ref.dtype)
        return carry
    lax.fori_loop(0, n // LANES, body, 0)


def _make_deg_kernel(nch):
    @functools.partial(
        pl.kernel,
        out_type=jax.ShapeDtypeStruct((NC, NPAD), jnp.float32),
        mesh=_MESH,
        scratch_types=[
            pltpu.VMEM((nch, K), jnp.int32),
            pltpu.VMEM((K,), jnp.float32),
            pltpu.VMEM((RPT,), jnp.float32),
            pltpu.VMEM_SHARED((NPAD,), jnp.float32),
        ],
        compiler_params=_PARAMS,
    )
    def deg_kernel(dst_hbm, out_hbm, idx_v, ones_v, zero_v, acc):
        cid = lax.axis_index("c")
        sid = lax.axis_index("s")
        wid = cid * NS + sid
        pltpu.sync_copy(dst_hbm.at[wid], idx_v)
        _fill1(ones_v, 1.0, K)
        _fill1(zero_v, 0.0, RPT)
        pltpu.sync_copy(zero_v, acc.at[pl.ds(sid * RPT, RPT)])
        plsc.subcore_barrier()

        def chunk(i, carry):
            pltpu.sync_copy(ones_v, acc.at[idx_v.at[i]], add=True)
            return carry

        lax.fori_loop(0, nch, chunk, 0)
        plsc.subcore_barrier()
        pltpu.sync_copy(acc.at[pl.ds(sid * RPT, RPT)],
                        out_hbm.at[cid, pl.ds(sid * RPT, RPT)])

    return deg_kernel


def _make_scatter_kernel(nch, width):
    @functools.partial(
        pl.kernel,
        out_type=jax.ShapeDtypeStruct((NC, NPAD, width), jnp.float32),
        mesh=_MESH,
        scratch_types=[
            pltpu.VMEM((nch, K), jnp.int32),
            pltpu.VMEM((nch, K), jnp.int32),
            pltpu.VMEM((2, K, width), jnp.float32),
            pltpu.VMEM((RPT, width), jnp.float32),
            pltpu.VMEM_SHARED((NPAD, width), jnp.float32),
            pltpu.SemaphoreType.DMA,
        ],
        compiler_params=_PARAMS,
    )
    def scatter_kernel(tab_hbm, src_hbm, dst_hbm, zero_hbm, out_hbm,
                       idx_s, idx_d, rows, zv, acc, sem):
        cid = lax.axis_index("c")
        sid = lax.axis_index("s")
        wid = cid * NS + sid
        pltpu.sync_copy(src_hbm.at[wid], idx_s)
        pltpu.sync_copy(dst_hbm.at[wid], idx_d)
        pltpu.sync_copy(zero_hbm, zv)
        pltpu.sync_copy(zv, acc.at[pl.ds(sid * RPT, RPT)])
        plsc.subcore_barrier()

        pltpu.async_copy(tab_hbm.at[idx_s.at[0]], rows.at[0], sem)

        def chunk(i, carry):
            cur = lax.rem(i, 2)
            pltpu.make_async_copy(tab_hbm.at[idx_s.at[i]], rows.at[cur],
                                  sem).wait()

            @pl.when(i + 1 < nch)
            def _():
                pltpu.async_copy(tab_hbm.at[idx_s.at[i + 1]],
                                 rows.at[1 - cur], sem)

            pltpu.sync_copy(rows.at[cur], acc.at[idx_d.at[i]], add=True)
            return carry

        lax.fori_loop(0, nch, chunk, 0)
        plsc.subcore_barrier()
        pltpu.sync_copy(acc.at[pl.ds(sid * RPT, RPT)],
                        out_hbm.at[cid, pl.ds(sid * RPT, RPT)])

    return scatter_kernel


def _tc_stats_body(x_ref, v_ref, gv_ref, bv_ref, bfe_ref, W1fe_ref, W1v_ref,
                   hw1_ref):
    xx = x_ref[...]
    n = xx.shape[0]
    h = hw1_ref.shape[1]
    mx = jnp.mean(xx, axis=0, keepdims=True)
    vx = jnp.mean((xx - mx) ** 2, axis=0, keepdims=True)
    v = v_ref[...]
    s = gv_ref[...] * v * lax.rsqrt(v * v * vx + 1e-5)
    o = bv_ref[...] - s * mx
    Kc = (jnp.dot(bfe_ref[...], W1fe_ref[...], preferred_element_type=jnp.float32)
          + jnp.dot(o, W1v_ref[...], preferred_element_type=jnp.float32))
    hw1_ref[:n, :] = (jnp.dot(xx * s, W1v_ref[...],
                              preferred_element_type=jnp.float32) + Kc)
    hw1_ref[n:, :] = jnp.zeros((hw1_ref.shape[0] - n, h), jnp.float32)


def _tc_g1_body(hw1_ref, d0_ref, d1_ref, g1_ref, dinv_ref):
    n = dinv_ref.shape[0]
    h = g1_ref.shape[1]
    deg = d0_ref[:n] + d1_ref[:n] + 1.0
    dinv = lax.rsqrt(deg)
    dinv_ref[...] = dinv
    g1_ref[:n, :] = hw1_ref[:n, :] * dinv
    g1_ref[n:, :] = jnp.zeros((g1_ref.shape[0] - n, h), jnp.float32)


def _tc_mid_body(a10_ref, a11_ref, g1_ref, dinv_ref, b1_ref, g2_ref):
    n = dinv_ref.shape[0]
    h = g2_ref.shape[1]
    dinv = dinv_ref[...]
    acc = a10_ref[:n] + a11_ref[:n] + g1_ref[:n]
    out1 = jnp.maximum(acc * dinv + b1_ref[...], 0.0)
    g2_ref[:n, :] = out1 * dinv
    g2_ref[n:, :] = jnp.zeros((g2_ref.shape[0] - n, h), jnp.float32)


def _tc_final_body(a20_ref, a21_ref, g2_ref, dinv_ref, W2_ref, b2_ref, out_ref):
    n, c = out_ref.shape
    hh = g2_ref.shape[1]
    pre = (a20_ref[:n, :hh] + a21_ref[:n, :hh] + g2_ref[:n, :hh]) * dinv_ref[...]
    h = jnp.dot(pre, W2_ref[...], preferred_element_type=jnp.float32) + b2_ref[...]
    m = jnp.max(h, axis=1, keepdims=True)
    lse = jnp.log(jnp.sum(jnp.exp(h - m), axis=1, keepdims=True))
    out_ref[...] = h - m - lse


def kernel(x, edge_index, feat_emb, val_emb, gamma, beta, W1, b1, W2, b2):
    N, D = x.shape
    E = edge_index.shape[1]
    FE = feat_emb.shape[1]
    CH = FE + val_emb.shape[1]
    H = W1.shape[1]
    C = W2.shape[1]

    nch = -(-E // (NW * K))
    EP = NW * K * nch
    pad = N + (jnp.arange(EP - E, dtype=jnp.int32) % (NPAD - N))
    src = jnp.concatenate([edge_index[0], pad]).reshape(NW, nch, K)
    dst = jnp.concatenate([edge_index[1], pad]).reshape(NW, nch, K)

    g5 = gamma.reshape(D, CH)
    b5 = beta.reshape(D, CH)
    W1r = W1.reshape(D, CH, H)
    gv = g5[:, FE].reshape(1, D)
    bv = b5[:, FE].reshape(1, D)
    v = val_emb[:, 0].reshape(1, D)
    W1v = W1r[:, FE, :]
    W1fe = W1r[:, :FE, :].reshape(FE * D, H)
    bfe = b5[:, :FE].reshape(1, FE * D)
    zero_h = jnp.zeros((RPT, H), jnp.float32)

    hw1 = pl.pallas_call(
        _tc_stats_body,
        out_shape=jax.ShapeDtypeStruct((NPAD, H), jnp.float32),
    )(x, v, gv, bv, bfe, W1fe, W1v)

    degp = _make_deg_kernel(nch)(dst)
    d0 = degp[0].reshape(NPAD, 1)
    d1 = degp[1].reshape(NPAD, 1)

    g1, dinv = pl.pallas_call(
        _tc_g1_body,
        out_shape=(jax.ShapeDtypeStruct((NPAD, H), jnp.float32),
                   jax.ShapeDtypeStruct((N, 1), jnp.float32)),
    )(hw1, d0, d1)

    acc1 = _make_scatter_kernel(nch, H)(g1, src, dst, zero_h)

    g2 = pl.pallas_call(
        _tc_mid_body,
        out_shape=jax.ShapeDtypeStruct((NPAD, H), jnp.float32),
    )(acc1[0], acc1[1], g1, dinv, b1.reshape(1, H))

    acc2 = _make_scatter_kernel(nch, H)(g2, src, dst, zero_h)

    out = pl.pallas_call(
        _tc_final_body,
        out_shape=jax.ShapeDtypeStruct((N, C), jnp.float32),
    )(acc2[0], acc2[1], g2, dinv, W2, b2.reshape(1, C))
    return out


# speedup vs baseline: 1.0456x; 1.0056x over previous
import functools

import jax
import jax.numpy as jnp
from jax import lax
from jax.experimental import pallas as pl
from jax.experimental.pallas import tpu as pltpu
from jax.experimental.pallas import tpu_sc as plsc

NC = 2
NS = 16
NW = NC * NS
LANES = 16
NPAD = 10240
RPT = NPAD // NS
K = 128

_MESH = plsc.VectorSubcoreMesh(core_axis_name="c", subcore_axis_name="s")
_PARAMS = pltpu.CompilerParams(use_tc_tiling_on_sc=False,
                               skip_device_barrier=True)


def _fill1(ref, val, n):
    def body(i, carry):
        ref[pl.ds(i * LANES, LANES)] = jnp.full((LANES,), val, ref.dtype)
        return carry
    lax.fori_loop(0, n // LANES, body, 0)


def _make_deg_kernel(nch):
    @functools.partial(
        pl.kernel,
        out_type=jax.ShapeDtypeStruct((NC, NPAD), jnp.float32),
        mesh=_MESH,
        scratch_types=[
            pltpu.VMEM((nch, K), jnp.int32),
            pltpu.VMEM((K,), jnp.float32),
            pltpu.VMEM((RPT,), jnp.float32),
            pltpu.VMEM_SHARED((NPAD,), jnp.float32),
        ],
        compiler_params=_PARAMS,
    )
    def deg_kernel(dst_hbm, out_hbm, idx_v, ones_v, zero_v, acc):
        cid = lax.axis_index("c")
        sid = lax.axis_index("s")
        wid = cid * NS + sid
        pltpu.sync_copy(dst_hbm.at[wid], idx_v)
        _fill1(ones_v, 1.0, K)
        _fill1(zero_v, 0.0, RPT)
        pltpu.sync_copy(zero_v, acc.at[pl.ds(sid * RPT, RPT)])
        plsc.subcore_barrier()

        def chunk(i, carry):
            pltpu.sync_copy(ones_v, acc.at[idx_v.at[i]], add=True)
            return carry

        lax.fori_loop(0, nch, chunk, 0)
        plsc.subcore_barrier()
        pltpu.sync_copy(acc.at[pl.ds(sid * RPT, RPT)],
                        out_hbm.at[cid, pl.ds(sid * RPT, RPT)])

    return deg_kernel


def _make_scatter_kernel(nch, width):
    @functools.partial(
        pl.kernel,
        out_type=jax.ShapeDtypeStruct((NC, NPAD, width), jnp.float32),
        mesh=_MESH,
        scratch_types=[
            pltpu.VMEM((nch, K), jnp.int32),
            pltpu.VMEM((nch, K), jnp.int32),
            pltpu.VMEM((2, K, width), jnp.float32),
            pltpu.VMEM((RPT, width), jnp.float32),
            pltpu.VMEM_SHARED((NPAD, width), jnp.float32),
            pltpu.SemaphoreType.DMA,
        ],
        compiler_params=_PARAMS,
    )
    def scatter_kernel(tab_hbm, src_hbm, dst_hbm, zero_hbm, out_hbm,
                       idx_s, idx_d, rows, zv, acc, sem):
        cid = lax.axis_index("c")
        sid = lax.axis_index("s")
        wid = cid * NS + sid
        pltpu.sync_copy(src_hbm.at[wid], idx_s)
        pltpu.sync_copy(dst_hbm.at[wid], idx_d)
        pltpu.sync_copy(zero_hbm, zv)
        pltpu.sync_copy(zv, acc.at[pl.ds(sid * RPT, RPT)])
        plsc.subcore_barrier()

        pltpu.async_copy(tab_hbm.at[idx_s.at[0]], rows.at[0], sem)

        def chunk(i, carry):
            cur = lax.rem(i, 2)
            pltpu.make_async_copy(tab_hbm.at[idx_s.at[i]], rows.at[cur],
                                  sem).wait()

            @pl.when(i + 1 < nch)
            def _():
                pltpu.async_copy(tab_hbm.at[idx_s.at[i + 1]],
                                 rows.at[1 - cur], sem)

            pltpu.sync_copy(rows.at[cur], acc.at[idx_d.at[i]], add=True)
            return carry

        lax.fori_loop(0, nch, chunk, 0)
        plsc.subcore_barrier()
        pltpu.sync_copy(acc.at[pl.ds(sid * RPT, RPT)],
                        out_hbm.at[cid, pl.ds(sid * RPT, RPT)])

    return scatter_kernel


def _tc_prep_body(x_ref, v_ref, gv_ref, bv_ref, bfe_ref, W1fe_ref, W1v_ref,
                  d0_ref, d1_ref, g1_ref, dinv_ref):
    xx = x_ref[...]
    n = xx.shape[0]
    h = g1_ref.shape[1]
    mx = jnp.mean(xx, axis=0, keepdims=True)
    vx = jnp.mean((xx - mx) ** 2, axis=0, keepdims=True)
    v = v_ref[...]
    s = gv_ref[...] * v * lax.rsqrt(v * v * vx + 1e-5)
    o = bv_ref[...] - s * mx
    Kc = (jnp.dot(bfe_ref[...], W1fe_ref[...], preferred_element_type=jnp.float32)
          + jnp.dot(o, W1v_ref[...], preferred_element_type=jnp.float32))
    hw1 = jnp.dot(xx * s, W1v_ref[...], preferred_element_type=jnp.float32) + Kc
    deg = d0_ref[:n] + d1_ref[:n] + 1.0
    dinv = lax.rsqrt(deg)
    dinv_ref[...] = dinv
    g1_ref[:n, :] = hw1 * dinv
    g1_ref[n:, :] = jnp.zeros((g1_ref.shape[0] - n, h), jnp.float32)


def _tc_mid_body(a10_ref, a11_ref, g1_ref, dinv_ref, b1_ref, g2_ref):
    n = dinv_ref.shape[0]
    h = g2_ref.shape[1]
    dinv = dinv_ref[...]
    acc = a10_ref[:n] + a11_ref[:n] + g1_ref[:n]
    out1 = jnp.maximum(acc * dinv + b1_ref[...], 0.0)
    g2_ref[:n, :] = out1 * dinv
    g2_ref[n:, :] = jnp.zeros((g2_ref.shape[0] - n, h), jnp.float32)


def _tc_final_body(a20_ref, a21_ref, g2_ref, dinv_ref, W2_ref, b2_ref, out_ref):
    n, c = out_ref.shape
    hh = g2_ref.shape[1]
    pre = (a20_ref[:n, :hh] + a21_ref[:n, :hh] + g2_ref[:n, :hh]) * dinv_ref[...]
    h = jnp.dot(pre, W2_ref[...], preferred_element_type=jnp.float32) + b2_ref[...]
    m = jnp.max(h, axis=1, keepdims=True)
    lse = jnp.log(jnp.sum(jnp.exp(h - m), axis=1, keepdims=True))
    out_ref[...] = h - m - lse


def kernel(x, edge_index, feat_emb, val_emb, gamma, beta, W1, b1, W2, b2):
    N, D = x.shape
    E = edge_index.shape[1]
    FE = feat_emb.shape[1]
    CH = FE + val_emb.shape[1]
    H = W1.shape[1]
    C = W2.shape[1]

    nch = -(-E // (NW * K))
    EP = NW * K * nch
    pad = N + (jnp.arange(EP - E, dtype=jnp.int32) % (NPAD - N))
    src = jnp.concatenate([edge_index[0], pad]).reshape(NW, nch, K)
    dst = jnp.concatenate([edge_index[1], pad]).reshape(NW, nch, K)

    g5 = gamma.reshape(D, CH)
    b5 = beta.reshape(D, CH)
    W1r = W1.reshape(D, CH, H)
    gv = g5[:, FE].reshape(1, D)
    bv = b5[:, FE].reshape(1, D)
    v = val_emb[:, 0].reshape(1, D)
    W1v = W1r[:, FE, :]
    W1fe = W1r[:, :FE, :].reshape(FE * D, H)
    bfe = b5[:, :FE].reshape(1, FE * D)
    zero_h = jnp.zeros((RPT, H), jnp.float32)

    degp = _make_deg_kernel(nch)(dst)
    d0 = degp[0].reshape(NPAD, 1)
    d1 = degp[1].reshape(NPAD, 1)

    g1, dinv = pl.pallas_call(
        _tc_prep_body,
        out_shape=(jax.ShapeDtypeStruct((NPAD, H), jnp.float32),
                   jax.ShapeDtypeStruct((N, 1), jnp.float32)),
    )(x, v, gv, bv, bfe, W1fe, W1v, d0, d1)

    acc1 = _make_scatter_kernel(nch, H)(g1, src, dst, zero_h)

    g2 = pl.pallas_call(
        _tc_mid_body,
        out_shape=jax.ShapeDtypeStruct((NPAD, H), jnp.float32),
    )(acc1[0], acc1[1], g1, dinv, b1.reshape(1, H))

    acc2 = _make_scatter_kernel(nch, H)(g2, src, dst, zero_h)

    out = pl.pallas_call(
        _tc_final_body,
        out_shape=jax.ShapeDtypeStruct((N, C), jnp.float32),
    )(acc2[0], acc2[1], g2, dinv, W2, b2.reshape(1, C))
    return out
